# Initial kernel scaffold; baseline (speedup 1.0000x reference)
import functools

import jax
import jax.numpy as jnp
from jax import lax
from jax.experimental import pallas as pl
from jax.experimental.pallas import tpu as pltpu
from jax.experimental.pallas import tpu_sc as plsc

N = 10000
E = 320000
D = 128
G = 64
NC, NS = 2, 16
NW = NC * NS
EPT = E // NW
CHUNK = 80
NCHUNK = EPT // CHUNK
RPT = N // NS

_sc_mesh = plsc.VectorSubcoreMesh(core_axis_name="c", subcore_axis_name="s")


@functools.partial(
    pl.kernel,
    out_type=jax.ShapeDtypeStruct((2 * N, 16), jnp.float32),
    mesh=_sc_mesh,
    scratch_types=[
        pltpu.VMEM((NCHUNK, CHUNK), jnp.int32),
        pltpu.VMEM((CHUNK, 16), jnp.float32),
        pltpu.VMEM((RPT, 16), jnp.float32),
        pltpu.VMEM_SHARED((N, 16), jnp.float32),
    ],
)
def _deg_kernel(dst_hbm, out_hbm, didx, ones_v, zbuf, acc):
    c = lax.axis_index("c")
    s = lax.axis_index("s")
    wid = c * NS + s

    ones16 = jnp.full((16,), 1.0, jnp.float32)
    zeros16 = jnp.zeros((16,), jnp.float32)

    def fill_ones(i, carry):
        ones_v[i, :] = ones16
        return carry

    lax.fori_loop(0, CHUNK, fill_ones, 0)

    def fill_zero(i, carry):
        zbuf[i, :] = zeros16
        return carry

    lax.fori_loop(0, RPT, fill_zero, 0)

    pltpu.sync_copy(zbuf, acc.at[pl.ds(s * RPT, RPT)])
    plsc.subcore_barrier()

    pltpu.sync_copy(dst_hbm.at[pl.ds(wid * NCHUNK, NCHUNK)], didx)

    def body(j, carry):
        pltpu.sync_copy(ones_v, acc.at[didx.at[j]], add=True)
        return carry

    lax.fori_loop(0, NCHUNK, body, 0)
    plsc.subcore_barrier()

    pltpu.sync_copy(
        acc.at[pl.ds(s * RPT, RPT)],
        out_hbm.at[pl.ds(c * N + s * RPT, RPT)],
    )


@functools.partial(
    pl.kernel,
    out_type=jax.ShapeDtypeStruct((2 * N, D), jnp.float32),
    mesh=_sc_mesh,
    scratch_types=[
        pltpu.VMEM((NCHUNK, CHUNK), jnp.int32),
        pltpu.VMEM((NCHUNK, CHUNK), jnp.int32),
        pltpu.VMEM((CHUNK, D), jnp.float32),
        pltpu.VMEM((CHUNK, D), jnp.float32),
        pltpu.VMEM((RPT // 5, D), jnp.float32),
        pltpu.VMEM_SHARED((N, D), jnp.float32),
        pltpu.SemaphoreType.DMA,
        pltpu.SemaphoreType.DMA,
    ],
)
def _prop_kernel(q_hbm, src_hbm, dst_hbm, out_hbm,
                 sidx, didx, rows_a, rows_b, zbuf, acc, sem_a, sem_b):
    c = lax.axis_index("c")
    s = lax.axis_index("s")
    wid = c * NS + s

    zeros16 = jnp.zeros((16,), jnp.float32)

    def fill_zero(i, carry):
        for k in range(D // 16):
            zbuf[i, pl.ds(k * 16, 16)] = zeros16
        return carry

    lax.fori_loop(0, RPT // 5, fill_zero, 0)
    for t in range(5):
        pltpu.sync_copy(zbuf, acc.at[pl.ds(s * RPT + t * (RPT // 5), RPT // 5)])
    plsc.subcore_barrier()

    pltpu.sync_copy(src_hbm.at[pl.ds(wid * NCHUNK, NCHUNK)], sidx)
    pltpu.sync_copy(dst_hbm.at[pl.ds(wid * NCHUNK, NCHUNK)], didx)

    pltpu.async_copy(q_hbm.at[sidx.at[0]], rows_a, sem_a)

    def body(j, carry):
        @pl.when(j % 2 == 0)
        def _even():
            pltpu.make_async_copy(q_hbm.at[sidx.at[j]], rows_a, sem_a).wait()

            @pl.when(j + 1 < NCHUNK)
            def _pf():
                pltpu.async_copy(q_hbm.at[sidx.at[j + 1]], rows_b, sem_b)

            pltpu.sync_copy(rows_a, acc.at[didx.at[j]], add=True)

        @pl.when(j % 2 == 1)
        def _odd():
            pltpu.make_async_copy(q_hbm.at[sidx.at[j]], rows_b, sem_b).wait()

            @pl.when(j + 1 < NCHUNK)
            def _pf():
                pltpu.async_copy(q_hbm.at[sidx.at[j + 1]], rows_a, sem_a)

            pltpu.sync_copy(rows_b, acc.at[didx.at[j]], add=True)

        return carry

    lax.fori_loop(0, NCHUNK, body, 0)
    plsc.subcore_barrier()

    pltpu.sync_copy(
        acc.at[pl.ds(s * RPT, RPT)],
        out_hbm.at[pl.ds(c * N + s * RPT, RPT)],
    )


def _dis(degp_ref):
    deg = degp_ref[0:N, 0:1] + degp_ref[N:2 * N, 0:1] + 1.0
    return lax.rsqrt(deg)


def _tc_first_body(degp_ref, x_ref, w_ref, q_ref):
    dis = _dis(degp_ref)
    q_ref[...] = dis * jnp.dot(x_ref[...], w_ref[...],
                               preferred_element_type=jnp.float32)


def _tc_mid_body(degp_ref, p_ref, q_ref, b_ref, w_ref, qo_ref):
    dis = _dis(degp_ref)
    h = dis * (p_ref[0:N, :] + p_ref[N:2 * N, :] + q_ref[...]) + b_ref[...]
    h = jnp.maximum(h, 0.0)
    qo_ref[...] = dis * jnp.dot(h, w_ref[...],
                                preferred_element_type=jnp.float32)


def _tc_final_body(degp_ref, p_ref, q_ref, b_ref, batch_ref, wl_ref, bl_ref,
                   out_ref):
    dis = _dis(degp_ref)
    h = dis * (p_ref[0:N, :] + p_ref[N:2 * N, :] + q_ref[...]) + b_ref[...]
    gid = lax.broadcasted_iota(jnp.int32, (G, N), 0)
    onehot = (gid == batch_ref[...]).astype(jnp.float32)
    sums = jnp.dot(onehot, h, preferred_element_type=jnp.float32)
    cnts = jnp.sum(onehot, axis=1, keepdims=True)
    g = sums / jnp.maximum(cnts, 1.0)
    logits = jnp.dot(g, wl_ref[...], preferred_element_type=jnp.float32)
    logits = logits + bl_ref[...]
    m = jnp.max(logits, axis=1, keepdims=True)
    lse = jnp.log(jnp.sum(jnp.exp(logits - m), axis=1, keepdims=True)) + m
    out_ref[...] = logits - lse


_tc_first = pl.pallas_call(
    _tc_first_body, out_shape=jax.ShapeDtypeStruct((N, D), jnp.float32))
_tc_mid = pl.pallas_call(
    _tc_mid_body, out_shape=jax.ShapeDtypeStruct((N, D), jnp.float32))
_tc_final = pl.pallas_call(
    _tc_final_body, out_shape=jax.ShapeDtypeStruct((G, 4), jnp.float32))


def kernel(x, edge_index, batch, W1, b1, W2, b2, W3, b3, Wl, bl):
    src = edge_index[0].astype(jnp.int32).reshape(E // CHUNK, CHUNK)
    dst = edge_index[1].astype(jnp.int32).reshape(E // CHUNK, CHUNK)
    batch2d = batch.astype(jnp.int32).reshape(1, N)

    degp = _deg_kernel(dst)
    q1 = _tc_first(degp, x, W1)
    p1 = _prop_kernel(q1, src, dst)
    q2 = _tc_mid(degp, p1, q1, b1.reshape(1, D), W2)
    p2 = _prop_kernel(q2, src, dst)
    q3 = _tc_mid(degp, p2, q2, b2.reshape(1, D), W3)
    p3 = _prop_kernel(q3, src, dst)
    return _tc_final(degp, p3, q3, b3.reshape(1, D), batch2d, Wl,
                     bl.reshape(1, 4))


# speedup vs baseline: 20.5044x; 20.5044x over previous
import functools

import jax
import jax.numpy as jnp
from jax import lax
from jax.experimental import pallas as pl
from jax.experimental.pallas import tpu as pltpu
from jax.experimental.pallas import tpu_sc as plsc

N = 10000
E = 320000
D = 128
G = 64
NC, NS = 2, 16
NW = NC * NS
EPT = E // NW
CHUNK = 80
NCHUNK = EPT // CHUNK
RPT = N // NS

_sc_mesh = plsc.VectorSubcoreMesh(core_axis_name="c", subcore_axis_name="s")


@functools.partial(
    pl.kernel,
    out_type=jax.ShapeDtypeStruct((NC, NS, RPT, 16), jnp.float32),
    mesh=_sc_mesh,
    scratch_types=[
        pltpu.VMEM((NCHUNK, CHUNK), jnp.int32),
        pltpu.VMEM((CHUNK, 16), jnp.float32),
        pltpu.VMEM((CHUNK, 16), jnp.float32),
        pltpu.VMEM_SHARED((N, 16), jnp.float32),
    ],
)
def _deg_kernel(dst_hbm, out_hbm, didx, ones_v, zbuf, acc):
    c = lax.axis_index("c")
    s = lax.axis_index("s")
    wid = c * NS + s

    ones16 = jnp.full((16,), 1.0, jnp.float32)
    zeros16 = jnp.zeros((16,), jnp.float32)

    def fill_ones(i, carry):
        ones_v[i, :] = ones16
        return carry

    lax.fori_loop(0, CHUNK, fill_ones, 0)

    def fill_zero(i, carry):
        zbuf[i, :] = zeros16
        return carry

    lax.fori_loop(0, CHUNK, fill_zero, 0)

    for t in range(7):
        pltpu.sync_copy(zbuf, acc.at[pl.ds(s * RPT + t * CHUNK, CHUNK)])
    pltpu.sync_copy(zbuf.at[pl.ds(0, RPT - 7 * CHUNK)],
                    acc.at[pl.ds(s * RPT + 7 * CHUNK, RPT - 7 * CHUNK)])
    plsc.subcore_barrier()

    pltpu.sync_copy(dst_hbm.at[wid], didx)

    def body(j, carry):
        pltpu.sync_copy(ones_v, acc.at[didx.at[j]], add=True)
        return carry

    lax.fori_loop(0, NCHUNK, body, 0)
    plsc.subcore_barrier()

    pltpu.sync_copy(acc.at[pl.ds(s * RPT, RPT)], out_hbm.at[c, s])


@functools.partial(
    pl.kernel,
    out_type=jax.ShapeDtypeStruct((NC, NS, RPT, D), jnp.float32),
    mesh=_sc_mesh,
    scratch_types=[
        pltpu.VMEM((EPT,), jnp.int32),
        pltpu.VMEM((NCHUNK, CHUNK), jnp.int32),
        pltpu.VMEM((CHUNK, D), jnp.float32),
        pltpu.VMEM((CHUNK, D), jnp.float32),
        pltpu.VMEM_SHARED((N, D), jnp.float32),
        pltpu.SemaphoreType.DMA,
        pltpu.SemaphoreType.DMA,
    ],
)
def _prop_kernel(q_hbm, src_flat_hbm, dst_hbm, out_hbm,
                 sidx, didx, rows_a, rows_b, acc, sem_a, sem_b):
    c = lax.axis_index("c")
    s = lax.axis_index("s")
    wid = c * NS + s

    zeros16 = jnp.zeros((16,), jnp.float32)

    def fill_zero(i, carry):
        for k in range(D // 16):
            rows_a[i, pl.ds(k * 16, 16)] = zeros16
        return carry

    lax.fori_loop(0, CHUNK, fill_zero, 0)
    for t in range(7):
        pltpu.sync_copy(rows_a, acc.at[pl.ds(s * RPT + t * CHUNK, CHUNK)])
    pltpu.sync_copy(rows_a.at[pl.ds(0, RPT - 7 * CHUNK)],
                    acc.at[pl.ds(s * RPT + 7 * CHUNK, RPT - 7 * CHUNK)])
    plsc.subcore_barrier()

    pltpu.sync_copy(src_flat_hbm.at[pl.ds(wid * EPT, EPT)], sidx)
    pltpu.sync_copy(dst_hbm.at[wid], didx)

    pltpu.async_copy(q_hbm.at[sidx.at[pl.ds(0, CHUNK)]], rows_a, sem_a)

    def body(j, carry):
        @pl.when(j % 2 == 0)
        def _even():
            pltpu.make_async_copy(
                q_hbm.at[sidx.at[pl.ds(j * CHUNK, CHUNK)]], rows_a,
                sem_a).wait()

            @pl.when(j + 1 < NCHUNK)
            def _pf():
                pltpu.async_copy(
                    q_hbm.at[sidx.at[pl.ds((j + 1) * CHUNK, CHUNK)]], rows_b,
                    sem_b)

            pltpu.sync_copy(rows_a, acc.at[didx.at[j]], add=True)

        @pl.when(j % 2 == 1)
        def _odd():
            pltpu.make_async_copy(
                q_hbm.at[sidx.at[pl.ds(j * CHUNK, CHUNK)]], rows_b,
                sem_b).wait()

            @pl.when(j + 1 < NCHUNK)
            def _pf():
                pltpu.async_copy(
                    q_hbm.at[sidx.at[pl.ds((j + 1) * CHUNK, CHUNK)]], rows_a,
                    sem_a)

            pltpu.sync_copy(rows_b, acc.at[didx.at[j]], add=True)

        return carry

    lax.fori_loop(0, NCHUNK, body, 0)
    plsc.subcore_barrier()

    pltpu.sync_copy(acc.at[pl.ds(s * RPT, RPT)], out_hbm.at[c, s])


def _dis(degp_ref):
    deg = degp_ref[0:N, 0:1] + degp_ref[N:2 * N, 0:1] + 1.0
    return lax.rsqrt(deg)


def _tc_first_body(degp_ref, x_ref, w_ref, q_ref):
    dis = _dis(degp_ref)
    q_ref[...] = dis * jnp.dot(x_ref[...], w_ref[...],
                               preferred_element_type=jnp.float32)


def _tc_mid_body(degp_ref, p_ref, q_ref, b_ref, w_ref, qo_ref):
    dis = _dis(degp_ref)
    h = dis * (p_ref[0:N, :] + p_ref[N:2 * N, :] + q_ref[...]) + b_ref[...]
    h = jnp.maximum(h, 0.0)
    qo_ref[...] = dis * jnp.dot(h, w_ref[...],
                                preferred_element_type=jnp.float32)


def _tc_final_body(degp_ref, p_ref, q_ref, b_ref, batch_ref, wl_ref, bl_ref,
                   out_ref):
    dis = _dis(degp_ref)
    h = dis * (p_ref[0:N, :] + p_ref[N:2 * N, :] + q_ref[...]) + b_ref[...]
    gid = lax.broadcasted_iota(jnp.int32, (G, N), 0)
    onehot = (gid == batch_ref[...]).astype(jnp.float32)
    sums = jnp.dot(onehot, h, preferred_element_type=jnp.float32)
    cnts = jnp.sum(onehot, axis=1, keepdims=True)
    g = sums / jnp.maximum(cnts, 1.0)
    logits = jnp.dot(g, wl_ref[...], preferred_element_type=jnp.float32)
    logits = logits + bl_ref[...]
    m = jnp.max(logits, axis=1, keepdims=True)
    lse = jnp.log(jnp.sum(jnp.exp(logits - m), axis=1, keepdims=True)) + m
    out_ref[...] = logits - lse


_tc_first = pl.pallas_call(
    _tc_first_body, out_shape=jax.ShapeDtypeStruct((N, D), jnp.float32))
_tc_mid = pl.pallas_call(
    _tc_mid_body, out_shape=jax.ShapeDtypeStruct((N, D), jnp.float32))
_tc_final = pl.pallas_call(
    _tc_final_body, out_shape=jax.ShapeDtypeStruct((G, 4), jnp.float32))


def kernel(x, edge_index, batch, W1, b1, W2, b2, W3, b3, Wl, bl):
    src_flat = edge_index[0].astype(jnp.int32)
    dst = edge_index[1].astype(jnp.int32).reshape(NW, NCHUNK, CHUNK)
    batch2d = batch.astype(jnp.int32).reshape(1, N)

    degp = _deg_kernel(dst).reshape(2 * N, 16)
    q1 = _tc_first(degp, x, W1)
    p1 = _prop_kernel(q1, src_flat, dst).reshape(2 * N, D)
    q2 = _tc_mid(degp, p1, q1, b1.reshape(1, D), W2)
    p2 = _prop_kernel(q2, src_flat, dst).reshape(2 * N, D)
    q3 = _tc_mid(degp, p2, q2, b2.reshape(1, D), W3)
    p3 = _prop_kernel(q3, src_flat, dst).reshape(2 * N, D)
    return _tc_final(degp, p3, q3, b3.reshape(1, D), batch2d, Wl,
                     bl.reshape(1, 4))
